# baseline (speedup 1.0000x reference)
import functools

import jax
import jax.numpy as jnp
from jax import lax
from jax.experimental import pallas as pl
from jax.experimental.pallas import tpu as pltpu
from jax.experimental.pallas import tpu_sc as plsc

B, N_E, D_R = 32, 128, 64
N_TRI = N_E * (N_E + 1) // 2
CH = 256
N_CH = (N_TRI + CH - 1) // CH
OUT_ROWS = N_TRI * D_R // 128


def _chunk_pieces(jc):
    lo, hi = jc * CH, min((jc + 1) * CH, N_TRI)
    pieces = []
    for r in range(N_E):
        off = r * (r + 1) // 2
        a, b_ = max(off, lo), min(off + r + 1, hi)
        if a < b_:
            pieces.append((r * N_E + (a - off), b_ - a, a - lo))
    return pieces


def _flatten_tri_sc(x):
    mesh = plsc.VectorSubcoreMesh(core_axis_name="c", subcore_axis_name="s")

    @functools.partial(
        pl.kernel,
        mesh=mesh,
        compiler_params=pltpu.CompilerParams(use_tc_tiling_on_sc=False),
        out_type=jax.ShapeDtypeStruct((B, OUT_ROWS, 128), jnp.float32),
        scratch_types=[
            pltpu.VMEM((2, CH, D_R), jnp.float32),
            pltpu.VMEM((2, CH // 2, 128), jnp.float32),
            pltpu.SemaphoreType.DMA,
            pltpu.SemaphoreType.DMA,
            pltpu.SemaphoreType.DMA,
            pltpu.SemaphoreType.DMA,
        ],
    )
    def k(in_hbm, out_hbm, s1, s2, f0, f1, d0, d1):
        wid = lax.axis_index("s") * 2 + lax.axis_index("c")
        fsem = (f0, f1)
        dsem = (d0, d1)

        def fill(jc):
            buf = jc % 2
            cps = []
            for (estart, cnt, pos) in _chunk_pieces(jc):
                cps.append(
                    pltpu.async_copy(
                        in_hbm.at[wid, pl.ds(estart, cnt)],
                        s1.at[buf, pl.ds(pos, cnt)],
                        fsem[buf],
                    )
                )
            return cps

        def bridge(jc):
            buf = jc % 2
            n_pairs = (min((jc + 1) * CH, N_TRI) - jc * CH) // 2

            def body(c, carry):
                for k4 in range(4):
                    s2[buf, c, pl.ds(16 * k4, 16)] = s1[
                        buf, 2 * c, pl.ds(16 * k4, 16)
                    ]
                    s2[buf, c, pl.ds(64 + 16 * k4, 16)] = s1[
                        buf, 2 * c + 1, pl.ds(16 * k4, 16)
                    ]
                return carry

            lax.fori_loop(0, n_pairs, body, 0)

        def drain(jc):
            buf = jc % 2
            rows = (min((jc + 1) * CH, N_TRI) - jc * CH) // 2
            return pltpu.async_copy(
                s2.at[buf, pl.ds(0, rows)],
                out_hbm.at[wid, pl.ds(jc * (CH // 2), rows)],
                dsem[buf],
            )

        pending_fills = {0: fill(0), 1: fill(1)}
        pending_drains = {}
        for jc in range(N_CH):
            for c in pending_fills.pop(jc):
                c.wait()
            if jc - 2 in pending_drains:
                pending_drains.pop(jc - 2).wait()
            bridge(jc)
            pending_drains[jc] = drain(jc)
            if jc + 2 < N_CH:
                pending_fills[jc + 2] = fill(jc + 2)
        for jc in sorted(pending_drains):
            pending_drains.pop(jc).wait()

    return k(x)


def kernel(inputs):
    table = inputs.reshape(B, N_E * N_E, D_R)
    out = _flatten_tri_sc(table)
    return out.reshape(B, N_TRI * D_R)
